# baseline (speedup 1.0000x reference)
import functools

import jax
import jax.numpy as jnp
from jax import lax
from jax.experimental import pallas as pl
from jax.experimental.pallas import tpu as pltpu
from jax.experimental.pallas import tpu_sc as plsc

NC = 2
NS = 16
NW = NC * NS
EB = 128
BN = 256
ZROWS = 128

_SC_PARAMS = pltpu.CompilerParams(
    use_tc_tiling_on_sc=False, needs_layout_passes=False
)

_MESH = plsc.VectorSubcoreMesh(
    core_axis_name="c", subcore_axis_name="s", num_cores=NC, num_subcores=NS
)


def _wid():
    return lax.axis_index("s") * NC + lax.axis_index("c")


def _zero_fill_1d(ref, n):
    def body(i, c):
        ref[pl.ds(i * 16, 16)] = jnp.zeros((16,), jnp.float32)
        return c

    lax.fori_loop(0, n // 16, body, None)


def _zero_fill_2d(ref, rows):
    def body(i, c):
        ref[i, pl.ds(0, 16)] = jnp.zeros((16,), jnp.float32)
        ref[i, pl.ds(16, 16)] = jnp.zeros((16,), jnp.float32)
        return c

    lax.fori_loop(0, rows, body, None)


def _sc_deg(col, ea, n_pad):
    e = col.shape[0]
    nb = e // EB
    nbt = -(-nb // NW)
    pt = n_pad // NS

    def body(col_hbm, ea_hbm, degp_hbm, colbuf, eabuf, zbuf, acc):
        cid = lax.axis_index("c")
        sid = lax.axis_index("s")
        wid = _wid()
        _zero_fill_1d(zbuf, pt)
        pltpu.sync_copy(zbuf, acc.at[pl.ds(sid * pt, pt)])
        plsc.subcore_barrier()

        def ebody(k, c):
            b = wid + NW * k

            @pl.when(b < nb)
            def _w():
                pltpu.sync_copy(col_hbm.at[pl.ds(b * EB, EB)], colbuf.at[0])
                pltpu.sync_copy(ea_hbm.at[pl.ds(b * EB, EB)], eabuf)
                pltpu.sync_copy(eabuf, acc.at[colbuf.at[0]], add=True)

            return c

        lax.fori_loop(0, nbt, ebody, None)
        plsc.subcore_barrier()
        pltpu.sync_copy(
            acc.at[pl.ds(sid * pt, pt)],
            degp_hbm.at[pl.ds(cid * n_pad + sid * pt, pt)],
        )

    run = pl.kernel(
        body,
        out_type=jax.ShapeDtypeStruct((NC * n_pad,), jnp.float32),
        mesh=_MESH,
        compiler_params=_SC_PARAMS,
        scratch_types=[
            pltpu.VMEM((1, EB), jnp.int32),
            pltpu.VMEM((EB,), jnp.float32),
            pltpu.VMEM((pt,), jnp.float32),
            pltpu.VMEM_SHARED((n_pad,), jnp.float32),
        ],
    )
    return run(col, ea).reshape(NC, n_pad)


def _sc_edge2(row, col, t, u, n_pad):
    e = row.shape[0]
    nb = e // EB
    nbt = -(-nb // NW)
    pt = n_pad // NS

    def body(row_hbm, col_hbm, t_hbm, u_hbm, ea2_hbm, degp_hbm,
             rowbuf, colbuf, ea2buf, tbuf, ubuf, zbuf, acc):
        cid = lax.axis_index("c")
        sid = lax.axis_index("s")
        wid = _wid()
        _zero_fill_1d(zbuf, pt)
        pltpu.sync_copy(zbuf, acc.at[pl.ds(sid * pt, pt)])
        pltpu.sync_copy(t_hbm, tbuf)
        pltpu.sync_copy(u_hbm, ubuf)
        plsc.subcore_barrier()

        def ebody(k, c):
            b = wid + NW * k

            @pl.when(b < nb)
            def _w():
                pltpu.sync_copy(row_hbm.at[pl.ds(b * EB, EB)], rowbuf)
                pltpu.sync_copy(col_hbm.at[pl.ds(b * EB, EB)], colbuf.at[0])

                def jbody(j, c2):
                    ri = rowbuf[pl.ds(j * 16, 16)]
                    ci = colbuf[0, pl.ds(j * 16, 16)]
                    tv = plsc.load_gather(tbuf, [ri])
                    uv = plsc.load_gather(ubuf, [ci])
                    ea2buf[pl.ds(j * 16, 16)] = jnp.maximum(tv + uv, 0.0)
                    return c2

                lax.fori_loop(0, EB // 16, jbody, None)
                pltpu.sync_copy(ea2buf, ea2_hbm.at[pl.ds(b * EB, EB)])
                pltpu.sync_copy(ea2buf, acc.at[colbuf.at[0]], add=True)

            return c

        lax.fori_loop(0, nbt, ebody, None)
        plsc.subcore_barrier()
        pltpu.sync_copy(
            acc.at[pl.ds(sid * pt, pt)],
            degp_hbm.at[pl.ds(cid * n_pad + sid * pt, pt)],
        )

    run = pl.kernel(
        body,
        out_type=(
            jax.ShapeDtypeStruct((e,), jnp.float32),
            jax.ShapeDtypeStruct((NC * n_pad,), jnp.float32),
        ),
        mesh=_MESH,
        compiler_params=_SC_PARAMS,
        scratch_types=[
            pltpu.VMEM((EB,), jnp.int32),
            pltpu.VMEM((1, EB), jnp.int32),
            pltpu.VMEM((EB,), jnp.float32),
            pltpu.VMEM((n_pad,), jnp.float32),
            pltpu.VMEM((n_pad,), jnp.float32),
            pltpu.VMEM((pt,), jnp.float32),
            pltpu.VMEM_SHARED((n_pad,), jnp.float32),
        ],
    )
    ea2, degp = run(row, col, t, u)
    return ea2, degp.reshape(NC, n_pad)


def _sc_agg(g2d, row, col, w, n_pad):
    f_total = g2d.shape[1] // 32
    fc = f_total // NC
    e = row.shape[0]
    nb = e // EB
    nbt = -(-nb // NS)
    pt = n_pad // NS
    nz = pt // ZROWS

    def body(g_hbm, row_hbm, col_hbm, w_hbm, agg_hbm,
             rowbuf, colbuf, wbuf, rows_v, zbuf, acc, sem):
        cid = lax.axis_index("c")
        sid = lax.axis_index("s")
        wid = _wid()
        _zero_fill_2d(zbuf, ZROWS)
        for j in range(fc):
            f = cid * fc + j

            def zb(i, c):
                pltpu.sync_copy(zbuf, acc.at[pl.ds(sid * pt + i * ZROWS, ZROWS)])
                return c

            lax.fori_loop(0, nz, zb, None)
            plsc.subcore_barrier()

            def ebody(k, c):
                b = sid + NS * k

                @pl.when(b < nb)
                def _w():
                    pltpu.sync_copy(row_hbm.at[pl.ds(b * EB, EB)], rowbuf)
                    pltpu.sync_copy(col_hbm.at[pl.ds(b * EB, EB)], colbuf.at[0])
                    pltpu.sync_copy(w_hbm.at[pl.ds(b * EB, EB)], wbuf)

                    def bias(j2, c3):
                        v = rowbuf[pl.ds(j2 * 16, 16)]
                        rowbuf[pl.ds(j2 * 16, 16)] = v + f * n_pad
                        return c3

                    lax.fori_loop(0, EB // 16, bias, None)
                    pltpu.async_copy(g_hbm.at[rowbuf], rows_v, sem).wait()

                    def sbody(j, c2):
                        wv = wbuf[pl.ds(j * 16, 16)]
                        for l in range(16):
                            ei = j * 16 + l
                            s = wv[l]
                            rows_v[ei, pl.ds(0, 16)] = rows_v[ei, pl.ds(0, 16)] * s
                            rows_v[ei, pl.ds(16, 16)] = rows_v[ei, pl.ds(16, 16)] * s
                        return c2

                    lax.fori_loop(0, EB // 16, sbody, None)
                    pltpu.sync_copy(rows_v, acc.at[colbuf.at[0]], add=True)

                return c

            lax.fori_loop(0, nbt, ebody, None)
            plsc.subcore_barrier()

            def cout(i, c):
                r0 = sid * pt + i * ZROWS
                pltpu.sync_copy(
                    acc.at[pl.ds(r0, ZROWS)],
                    agg_hbm.at[pl.ds(f * n_pad + r0, ZROWS)],
                )
                return c

            lax.fori_loop(0, nz, cout, None)
            plsc.subcore_barrier()

    run = pl.kernel(
        body,
        out_type=jax.ShapeDtypeStruct((f_total * n_pad, 32), jnp.float32),
        mesh=_MESH,
        compiler_params=_SC_PARAMS,
        scratch_types=[
            pltpu.VMEM((EB,), jnp.int32),
            pltpu.VMEM((1, EB), jnp.int32),
            pltpu.VMEM((EB,), jnp.float32),
            pltpu.VMEM((EB, 32), jnp.float32),
            pltpu.VMEM((ZROWS, 32), jnp.float32),
            pltpu.VMEM_SHARED((n_pad, 32), jnp.float32),
            pltpu.SemaphoreType.DMA,
        ],
    )
    g_sc = g2d.reshape(n_pad, f_total, 32).transpose(1, 0, 2).reshape(
        f_total * n_pad, 32
    )
    agg_sc = run(g_sc, row, col, w)
    return agg_sc.reshape(f_total, n_pad, 32).transpose(1, 0, 2).reshape(
        n_pad, f_total * 32
    )


def _tc_dis(degp, n_pad):
    blk = 1024

    def body(dp_ref, dis_ref):
        d = 1.0 + dp_ref[0, :] + dp_ref[1, :]
        dis_ref[...] = lax.rsqrt(d)[:, None]

    return pl.pallas_call(
        body,
        grid=(n_pad // blk,),
        in_specs=[pl.BlockSpec((NC, blk), lambda i: (0, i))],
        out_specs=pl.BlockSpec((blk, 1), lambda i: (i, 0)),
        out_shape=jax.ShapeDtypeStruct((n_pad, 1), jnp.float32),
    )(degp)


def _tc_mm_scale(x, w, dis, n_pad):
    n = x.shape[0]
    ngrid = -(-n // BN)
    d_in = x.shape[1]
    d_out = w.shape[1]

    def body(x_ref, w_ref, dis_ref, g_ref):
        h = jnp.dot(x_ref[...], w_ref[...], preferred_element_type=jnp.float32)
        g_ref[...] = h * dis_ref[...]

    return pl.pallas_call(
        body,
        grid=(ngrid,),
        in_specs=[
            pl.BlockSpec((BN, d_in), lambda i: (i, 0)),
            pl.BlockSpec((d_in, d_out), lambda i: (0, 0)),
            pl.BlockSpec((BN, 1), lambda i: (i, 0)),
        ],
        out_specs=pl.BlockSpec((BN, d_out), lambda i: (i, 0)),
        out_shape=jax.ShapeDtypeStruct((n_pad, d_out), jnp.float32),
    )(x, w, dis)


def _tc_out1(agg, g, dis, b1, w2, wm2, bm, n, n_pad):
    ngrid = -(-n // BN)
    dh = g.shape[1]
    d2 = w2.shape[1]

    def body(agg_ref, g_ref, dis_ref, b1_ref, w2_ref, wm_ref, bm_ref,
             h2_ref, t_ref, u_ref):
        x = agg_ref[...] + g_ref[...]
        x = jnp.maximum(b1_ref[...] + dis_ref[...] * x, 0.0)
        h2_ref[...] = jnp.dot(x, w2_ref[...], preferred_element_type=jnp.float32)
        tu = jnp.dot(x, wm_ref[...], preferred_element_type=jnp.float32)
        t_ref[...] = tu[:, 0:1] + bm_ref[...]
        u_ref[...] = tu[:, 1:2]

    return pl.pallas_call(
        body,
        grid=(ngrid,),
        in_specs=[
            pl.BlockSpec((BN, dh), lambda i: (i, 0)),
            pl.BlockSpec((BN, dh), lambda i: (i, 0)),
            pl.BlockSpec((BN, 1), lambda i: (i, 0)),
            pl.BlockSpec((1, dh), lambda i: (0, 0)),
            pl.BlockSpec((dh, d2), lambda i: (0, 0)),
            pl.BlockSpec((dh, 2), lambda i: (0, 0)),
            pl.BlockSpec((1, 1), lambda i: (0, 0)),
        ],
        out_specs=[
            pl.BlockSpec((BN, d2), lambda i: (i, 0)),
            pl.BlockSpec((BN, 1), lambda i: (i, 0)),
            pl.BlockSpec((BN, 1), lambda i: (i, 0)),
        ],
        out_shape=[
            jax.ShapeDtypeStruct((n_pad, d2), jnp.float32),
            jax.ShapeDtypeStruct((n_pad, 1), jnp.float32),
            jax.ShapeDtypeStruct((n_pad, 1), jnp.float32),
        ],
    )(agg, g, dis, b1, w2, wm2, bm)


def _tc_dis_scale(degp, h2, n, n_pad):
    ngrid = -(-n // BN)
    d2 = h2.shape[1]

    def body(dp_ref, h2_ref, g2_ref, dis_ref):
        d = 1.0 + dp_ref[0, :] + dp_ref[1, :]
        dis = lax.rsqrt(d)[:, None]
        dis_ref[...] = dis
        g2_ref[...] = h2_ref[...] * dis

    return pl.pallas_call(
        body,
        grid=(ngrid,),
        in_specs=[
            pl.BlockSpec((NC, BN), lambda i: (0, i)),
            pl.BlockSpec((BN, d2), lambda i: (i, 0)),
        ],
        out_specs=[
            pl.BlockSpec((BN, d2), lambda i: (i, 0)),
            pl.BlockSpec((BN, 1), lambda i: (i, 0)),
        ],
        out_shape=[
            jax.ShapeDtypeStruct((n_pad, d2), jnp.float32),
            jax.ShapeDtypeStruct((n_pad, 1), jnp.float32),
        ],
    )(degp, h2)


def _tc_out2(agg2, g2, dis2, b2, n, n_pad):
    d2 = g2.shape[1]

    def body(agg_ref, g_ref, dis_ref, b2_ref, out_ref):
        y = agg_ref[...] + g_ref[...]
        out_ref[...] = b2_ref[...] + dis_ref[...] * y

    return pl.pallas_call(
        body,
        grid=(-(-n // BN),),
        in_specs=[
            pl.BlockSpec((BN, d2), lambda i: (i, 0)),
            pl.BlockSpec((BN, d2), lambda i: (i, 0)),
            pl.BlockSpec((BN, 1), lambda i: (i, 0)),
            pl.BlockSpec((1, d2), lambda i: (0, 0)),
        ],
        out_specs=pl.BlockSpec((BN, d2), lambda i: (i, 0)),
        out_shape=jax.ShapeDtypeStruct((n, d2), jnp.float32),
    )(agg2, g2, dis2, b2)


def kernel(node_attr, edge_attr, edge_index, coords, frame, W1, b1, W2, b2, Wm, bm):
    n = node_attr.shape[0]
    dh = W1.shape[1]
    e = edge_attr.shape[0]
    row = edge_index[0]
    col = edge_index[1]
    ea = edge_attr.reshape(e)
    unit = NS * ZROWS
    n_pad = -(-n // unit) * unit

    wm2 = jnp.concatenate([Wm[:dh], Wm[dh:]], axis=1)

    degp1 = _sc_deg(col, ea, n_pad)
    dis1 = _tc_dis(degp1, n_pad)
    g1 = _tc_mm_scale(node_attr, W1, dis1, n_pad)
    agg1 = _sc_agg(g1, row, col, ea, n_pad)
    h2, t, u = _tc_out1(agg1, g1, dis1, b1.reshape(1, dh), W2, wm2,
                        bm.reshape(1, 1), n, n_pad)
    ea2, degp2 = _sc_edge2(row, col, t.reshape(n_pad), u.reshape(n_pad), n_pad)
    g2, dis2 = _tc_dis_scale(degp2, h2, n, n_pad)
    agg2 = _sc_agg(g2, row, col, ea2, n_pad)
    return _tc_out2(agg2, g2, dis2, b2.reshape(1, -1), n, n_pad)


# speedup vs baseline: 1.2464x; 1.2464x over previous
import functools

import jax
import jax.numpy as jnp
from jax import lax
from jax.experimental import pallas as pl
from jax.experimental.pallas import tpu as pltpu
from jax.experimental.pallas import tpu_sc as plsc

NC = 2
NS = 16
NW = NC * NS
EB = 128
BN = 256
ZROWS = 128

_SC_PARAMS = pltpu.CompilerParams(
    use_tc_tiling_on_sc=False, needs_layout_passes=False
)

_MESH = plsc.VectorSubcoreMesh(
    core_axis_name="c", subcore_axis_name="s", num_cores=NC, num_subcores=NS
)


def _wid():
    return lax.axis_index("s") * NC + lax.axis_index("c")


def _zero_fill_1d(ref, n):
    def body(i, c):
        ref[pl.ds(i * 16, 16)] = jnp.zeros((16,), jnp.float32)
        return c

    lax.fori_loop(0, n // 16, body, None)


def _zero_fill_2d(ref, rows):
    def body(i, c):
        ref[i, pl.ds(0, 16)] = jnp.zeros((16,), jnp.float32)
        ref[i, pl.ds(16, 16)] = jnp.zeros((16,), jnp.float32)
        return c

    lax.fori_loop(0, rows, body, None)


def _sc_deg(col, ea, n_pad):
    e = col.shape[0]
    nb = e // EB
    nbt = -(-nb // NW)
    pt = n_pad // NS

    def body(col_hbm, ea_hbm, degp_hbm, colbuf, eabuf, zbuf, acc):
        cid = lax.axis_index("c")
        sid = lax.axis_index("s")
        wid = _wid()
        _zero_fill_1d(zbuf, pt)
        pltpu.sync_copy(zbuf, acc.at[pl.ds(sid * pt, pt)])
        plsc.subcore_barrier()

        def ebody(k, c):
            b = wid + NW * k

            @pl.when(b < nb)
            def _w():
                pltpu.sync_copy(col_hbm.at[pl.ds(b * EB, EB)], colbuf.at[0])
                pltpu.sync_copy(ea_hbm.at[pl.ds(b * EB, EB)], eabuf)
                pltpu.sync_copy(eabuf, acc.at[colbuf.at[0]], add=True)

            return c

        lax.fori_loop(0, nbt, ebody, None)
        plsc.subcore_barrier()
        pltpu.sync_copy(
            acc.at[pl.ds(sid * pt, pt)],
            degp_hbm.at[pl.ds(cid * n_pad + sid * pt, pt)],
        )

    run = pl.kernel(
        body,
        out_type=jax.ShapeDtypeStruct((NC * n_pad,), jnp.float32),
        mesh=_MESH,
        compiler_params=_SC_PARAMS,
        scratch_types=[
            pltpu.VMEM((1, EB), jnp.int32),
            pltpu.VMEM((EB,), jnp.float32),
            pltpu.VMEM((pt,), jnp.float32),
            pltpu.VMEM_SHARED((n_pad,), jnp.float32),
        ],
    )
    return run(col, ea).reshape(NC, n_pad)


def _sc_edge2(row, col, t, u, n_pad):
    e = row.shape[0]
    nb = e // EB
    nbt = -(-nb // NW)
    pt = n_pad // NS

    def body(row_hbm, col_hbm, t_hbm, u_hbm, ea2_hbm, degp_hbm,
             rowbuf, colbuf, ea2buf, tbuf, ubuf, zbuf, acc):
        cid = lax.axis_index("c")
        sid = lax.axis_index("s")
        wid = _wid()
        _zero_fill_1d(zbuf, pt)
        pltpu.sync_copy(zbuf, acc.at[pl.ds(sid * pt, pt)])
        pltpu.sync_copy(t_hbm, tbuf)
        pltpu.sync_copy(u_hbm, ubuf)
        plsc.subcore_barrier()

        def ebody(k, c):
            b = wid + NW * k

            @pl.when(b < nb)
            def _w():
                pltpu.sync_copy(row_hbm.at[pl.ds(b * EB, EB)], rowbuf)
                pltpu.sync_copy(col_hbm.at[pl.ds(b * EB, EB)], colbuf.at[0])

                def jbody(j, c2):
                    ri = rowbuf[pl.ds(j * 16, 16)]
                    ci = colbuf[0, pl.ds(j * 16, 16)]
                    tv = plsc.load_gather(tbuf, [ri])
                    uv = plsc.load_gather(ubuf, [ci])
                    ea2buf[pl.ds(j * 16, 16)] = jnp.maximum(tv + uv, 0.0)
                    return c2

                lax.fori_loop(0, EB // 16, jbody, None)
                pltpu.sync_copy(ea2buf, ea2_hbm.at[pl.ds(b * EB, EB)])
                pltpu.sync_copy(ea2buf, acc.at[colbuf.at[0]], add=True)

            return c

        lax.fori_loop(0, nbt, ebody, None)
        plsc.subcore_barrier()
        pltpu.sync_copy(
            acc.at[pl.ds(sid * pt, pt)],
            degp_hbm.at[pl.ds(cid * n_pad + sid * pt, pt)],
        )

    run = pl.kernel(
        body,
        out_type=(
            jax.ShapeDtypeStruct((e,), jnp.float32),
            jax.ShapeDtypeStruct((NC * n_pad,), jnp.float32),
        ),
        mesh=_MESH,
        compiler_params=_SC_PARAMS,
        scratch_types=[
            pltpu.VMEM((EB,), jnp.int32),
            pltpu.VMEM((1, EB), jnp.int32),
            pltpu.VMEM((EB,), jnp.float32),
            pltpu.VMEM((n_pad,), jnp.float32),
            pltpu.VMEM((n_pad,), jnp.float32),
            pltpu.VMEM((pt,), jnp.float32),
            pltpu.VMEM_SHARED((n_pad,), jnp.float32),
        ],
    )
    ea2, degp = run(row, col, t, u)
    return ea2, degp.reshape(NC, n_pad)


GEB = 5


def _sc_agg(g2d, row, col, w, n_pad):
    f_total = g2d.shape[1] // 32
    fc = f_total // NC
    e = row.shape[0]
    nb = e // EB
    ng = nb // GEB
    assert ng * GEB == nb
    ngt = -(-ng // NS)
    pt = n_pad // NS
    nz = pt // ZROWS

    def body(g_hbm, row_hbm, col_hbm, w_hbm, agg_hbm,
             rowbuf, colbuf, wbuf, rows_v, zbuf, acc, sem_g, sem_s):
        cid = lax.axis_index("c")
        sid = lax.axis_index("s")
        _zero_fill_2d(zbuf, ZROWS)
        for j in range(fc):
            f = cid * fc + j

            def zb(i, c):
                pltpu.sync_copy(zbuf, acc.at[pl.ds(sid * pt + i * ZROWS, ZROWS)])
                return c

            lax.fori_loop(0, nz, zb, None)
            plsc.subcore_barrier()

            def ebody(m, c):
                gg = sid + NS * m

                @pl.when(gg < ng)
                def _w():
                    pltpu.sync_copy(row_hbm.at[pl.ds(gg * GEB, GEB)], rowbuf)
                    pltpu.sync_copy(col_hbm.at[pl.ds(gg * GEB, GEB)], colbuf)
                    pltpu.sync_copy(w_hbm.at[pl.ds(gg * GEB, GEB)], wbuf)

                    def bias(i, c3):
                        jj = i // 8
                        ii = i % 8
                        v = rowbuf[jj, pl.ds(ii * 16, 16)]
                        rowbuf[jj, pl.ds(ii * 16, 16)] = v + f * n_pad
                        return c3

                    lax.fori_loop(0, GEB * 8, bias, None)
                    descs = []
                    for bj in range(GEB):
                        descs.append(pltpu.async_copy(
                            g_hbm.at[rowbuf.at[bj]],
                            rows_v.at[pl.ds(bj * EB, EB)], sem_g))
                    for d in descs:
                        d.wait()

                    def sbody(j16, c2):
                        jj = j16 // 8
                        ii = j16 % 8
                        wv = wbuf[jj, pl.ds(ii * 16, 16)]
                        for l in range(16):
                            ei = jj * EB + ii * 16 + l
                            s = wv[l]
                            rows_v[ei, pl.ds(0, 16)] = rows_v[ei, pl.ds(0, 16)] * s
                            rows_v[ei, pl.ds(16, 16)] = (
                                rows_v[ei, pl.ds(16, 16)] * s)
                        return c2

                    lax.fori_loop(0, GEB * 8, sbody, None)
                    sdescs = []
                    for bj in range(GEB):
                        sdescs.append(pltpu.async_copy(
                            rows_v.at[pl.ds(bj * EB, EB)],
                            acc.at[colbuf.at[bj]], sem_s, add=True))
                    for d in sdescs:
                        d.wait()

                return c

            lax.fori_loop(0, ngt, ebody, None)
            plsc.subcore_barrier()

            def cout(i, c):
                r0 = sid * pt + i * ZROWS
                pltpu.sync_copy(
                    acc.at[pl.ds(r0, ZROWS)],
                    agg_hbm.at[pl.ds(f * n_pad + r0, ZROWS)],
                )
                return c

            lax.fori_loop(0, nz, cout, None)
            plsc.subcore_barrier()

    run = pl.kernel(
        body,
        out_type=jax.ShapeDtypeStruct((f_total * n_pad, 32), jnp.float32),
        mesh=_MESH,
        compiler_params=_SC_PARAMS,
        scratch_types=[
            pltpu.VMEM((GEB, EB), jnp.int32),
            pltpu.VMEM((GEB, EB), jnp.int32),
            pltpu.VMEM((GEB, EB), jnp.float32),
            pltpu.VMEM((GEB * EB, 32), jnp.float32),
            pltpu.VMEM((ZROWS, 32), jnp.float32),
            pltpu.VMEM_SHARED((n_pad, 32), jnp.float32),
            pltpu.SemaphoreType.DMA,
            pltpu.SemaphoreType.DMA,
        ],
    )
    g_sc = g2d.reshape(n_pad, f_total, 32).transpose(1, 0, 2).reshape(
        f_total * n_pad, 32
    )
    agg_sc = run(g_sc, row.reshape(nb, EB), col.reshape(nb, EB),
                 w.reshape(nb, EB))
    return agg_sc.reshape(f_total, n_pad, 32).transpose(1, 0, 2).reshape(
        n_pad, f_total * 32
    )


def _tc_dis(degp, n_pad):
    blk = 1024

    def body(dp_ref, dis_ref):
        d = 1.0 + dp_ref[0, :] + dp_ref[1, :]
        dis_ref[...] = lax.rsqrt(d)[:, None]

    return pl.pallas_call(
        body,
        grid=(n_pad // blk,),
        in_specs=[pl.BlockSpec((NC, blk), lambda i: (0, i))],
        out_specs=pl.BlockSpec((blk, 1), lambda i: (i, 0)),
        out_shape=jax.ShapeDtypeStruct((n_pad, 1), jnp.float32),
    )(degp)


def _tc_mm_scale(x, w, dis, n_pad):
    n = x.shape[0]
    ngrid = -(-n // BN)
    d_in = x.shape[1]
    d_out = w.shape[1]

    def body(x_ref, w_ref, dis_ref, g_ref):
        h = jnp.dot(x_ref[...], w_ref[...], preferred_element_type=jnp.float32)
        g_ref[...] = h * dis_ref[...]

    return pl.pallas_call(
        body,
        grid=(ngrid,),
        in_specs=[
            pl.BlockSpec((BN, d_in), lambda i: (i, 0)),
            pl.BlockSpec((d_in, d_out), lambda i: (0, 0)),
            pl.BlockSpec((BN, 1), lambda i: (i, 0)),
        ],
        out_specs=pl.BlockSpec((BN, d_out), lambda i: (i, 0)),
        out_shape=jax.ShapeDtypeStruct((n_pad, d_out), jnp.float32),
    )(x, w, dis)


def _tc_out1(agg, g, dis, b1, w2, wm2, bm, n, n_pad):
    ngrid = -(-n // BN)
    dh = g.shape[1]
    d2 = w2.shape[1]

    def body(agg_ref, g_ref, dis_ref, b1_ref, w2_ref, wm_ref, bm_ref,
             h2_ref, t_ref, u_ref):
        x = agg_ref[...] + g_ref[...]
        x = jnp.maximum(b1_ref[...] + dis_ref[...] * x, 0.0)
        h2_ref[...] = jnp.dot(x, w2_ref[...], preferred_element_type=jnp.float32)
        tu = jnp.dot(x, wm_ref[...], preferred_element_type=jnp.float32)
        t_ref[...] = tu[:, 0:1] + bm_ref[...]
        u_ref[...] = tu[:, 1:2]

    return pl.pallas_call(
        body,
        grid=(ngrid,),
        in_specs=[
            pl.BlockSpec((BN, dh), lambda i: (i, 0)),
            pl.BlockSpec((BN, dh), lambda i: (i, 0)),
            pl.BlockSpec((BN, 1), lambda i: (i, 0)),
            pl.BlockSpec((1, dh), lambda i: (0, 0)),
            pl.BlockSpec((dh, d2), lambda i: (0, 0)),
            pl.BlockSpec((dh, 2), lambda i: (0, 0)),
            pl.BlockSpec((1, 1), lambda i: (0, 0)),
        ],
        out_specs=[
            pl.BlockSpec((BN, d2), lambda i: (i, 0)),
            pl.BlockSpec((BN, 1), lambda i: (i, 0)),
            pl.BlockSpec((BN, 1), lambda i: (i, 0)),
        ],
        out_shape=[
            jax.ShapeDtypeStruct((n_pad, d2), jnp.float32),
            jax.ShapeDtypeStruct((n_pad, 1), jnp.float32),
            jax.ShapeDtypeStruct((n_pad, 1), jnp.float32),
        ],
    )(agg, g, dis, b1, w2, wm2, bm)


def _tc_dis_scale(degp, h2, n, n_pad):
    ngrid = -(-n // BN)
    d2 = h2.shape[1]

    def body(dp_ref, h2_ref, g2_ref, dis_ref):
        d = 1.0 + dp_ref[0, :] + dp_ref[1, :]
        dis = lax.rsqrt(d)[:, None]
        dis_ref[...] = dis
        g2_ref[...] = h2_ref[...] * dis

    return pl.pallas_call(
        body,
        grid=(ngrid,),
        in_specs=[
            pl.BlockSpec((NC, BN), lambda i: (0, i)),
            pl.BlockSpec((BN, d2), lambda i: (i, 0)),
        ],
        out_specs=[
            pl.BlockSpec((BN, d2), lambda i: (i, 0)),
            pl.BlockSpec((BN, 1), lambda i: (i, 0)),
        ],
        out_shape=[
            jax.ShapeDtypeStruct((n_pad, d2), jnp.float32),
            jax.ShapeDtypeStruct((n_pad, 1), jnp.float32),
        ],
    )(degp, h2)


def _tc_out2(agg2, g2, dis2, b2, n, n_pad):
    d2 = g2.shape[1]

    def body(agg_ref, g_ref, dis_ref, b2_ref, out_ref):
        y = agg_ref[...] + g_ref[...]
        out_ref[...] = b2_ref[...] + dis_ref[...] * y

    return pl.pallas_call(
        body,
        grid=(-(-n // BN),),
        in_specs=[
            pl.BlockSpec((BN, d2), lambda i: (i, 0)),
            pl.BlockSpec((BN, d2), lambda i: (i, 0)),
            pl.BlockSpec((BN, 1), lambda i: (i, 0)),
            pl.BlockSpec((1, d2), lambda i: (0, 0)),
        ],
        out_specs=pl.BlockSpec((BN, d2), lambda i: (i, 0)),
        out_shape=jax.ShapeDtypeStruct((n, d2), jnp.float32),
    )(agg2, g2, dis2, b2)


def kernel(node_attr, edge_attr, edge_index, coords, frame, W1, b1, W2, b2, Wm, bm):
    n = node_attr.shape[0]
    dh = W1.shape[1]
    e = edge_attr.shape[0]
    row = edge_index[0]
    col = edge_index[1]
    ea = edge_attr.reshape(e)
    unit = NS * ZROWS
    n_pad = -(-n // unit) * unit

    wm2 = jnp.concatenate([Wm[:dh], Wm[dh:]], axis=1)

    degp1 = _sc_deg(col, ea, n_pad)
    dis1 = _tc_dis(degp1, n_pad)
    g1 = _tc_mm_scale(node_attr, W1, dis1, n_pad)
    agg1 = _sc_agg(g1, row, col, ea, n_pad)
    h2, t, u = _tc_out1(agg1, g1, dis1, b1.reshape(1, dh), W2, wm2,
                        bm.reshape(1, 1), n, n_pad)
    ea2, degp2 = _sc_edge2(row, col, t.reshape(n_pad), u.reshape(n_pad), n_pad)
    g2, dis2 = _tc_dis_scale(degp2, h2, n, n_pad)
    agg2 = _sc_agg(g2, row, col, ea2, n_pad)
    return _tc_out2(agg2, g2, dis2, b2.reshape(1, -1), n, n_pad)
